# baseline (speedup 1.0000x reference)
import jax
import jax.numpy as jnp
from jax import lax
from jax.experimental import pallas as pl
from jax.experimental.pallas import tpu as pltpu
from jax.experimental.pallas import tpu_sc as plsc

N = 320000
D = 128
S = 10000

NC = 2
NS = 16
HALF = S // NC
CHUNK = 256
BROWS = CHUNK // 128
NCHUNKS = N // CHUNK
NBROWS = N // 128
ACC_R = 5120
TRASH = HALF
SCANQ = NBROWS // NS
SCANR = NBROWS - SCANQ * NS
DIVQ = 312


def _sc_body(x_hbm, b_hbm, out_hbm,
             buf_a, buf_b, rmp_a, rmp_b, ones_v, zcnt,
             sbuf, cntb, redv, pubv,
             acc_sh, cnt_sh, stage_sh, sem_ax, sem_bx,
             sem_ao, sem_bo):
    c = lax.axis_index("c")
    s = lax.axis_index("s")
    z16 = jnp.zeros((16,), jnp.float32)
    lane = lax.iota(jnp.int32, 16)

    for j in range(8):
        ones_v[pl.ds(j * 16, 16)] = jnp.ones((16,), jnp.float32)

    def zrow(i, _):
        for j in range(8):
            buf_a[i, pl.ds(j * 16, 16)] = z16
        return 0
    lax.fori_loop(0, 64, zrow, 0)

    def zc(i, _):
        zcnt[pl.ds(i * 16, 16)] = z16
        return 0
    lax.fori_loop(0, 64, zc, 0)

    rowbase = s * SCANQ + jnp.minimum(s, SCANR)
    pltpu.sync_copy(b_hbm.at[pl.ds(rowbase, SCANQ)], sbuf.at[pl.ds(0, SCANQ)])

    @pl.when(s < SCANR)
    def _extra_row():
        pltpu.sync_copy(b_hbm.at[pl.ds(rowbase + SCANQ, 1)],
                        sbuf.at[pl.ds(SCANQ, 1)])

    def scan_row(r, carry):
        fhi, fge = carry
        g = rowbase + r
        vhi = sbuf[r, pl.ds(112, 16)]
        vlo = sbuf[r, pl.ds(0, 16)]
        has_hi = jnp.any(vhi >= HALF)
        all_ge = jnp.all(vlo >= HALF)
        fhi = jnp.where(has_hi & (g < fhi), g, fhi)
        fge = jnp.where(all_ge & (g < fge), g, fge)
        return fhi, fge

    nrows = SCANQ + jnp.where(s < SCANR, 1, 0)
    fhi, fge = lax.fori_loop(0, nrows, scan_row,
                             (jnp.int32(NBROWS), jnp.int32(NBROWS)))
    pubv[...] = jnp.where(lane == 0, fhi, jnp.where(lane == 1, fge, NBROWS))
    pltpu.sync_copy(pubv, stage_sh.at[s])

    for b in range(5):
        pltpu.sync_copy(buf_a.at[pl.ds(0, 64)],
                        acc_sh.at[pl.ds(s * 320 + b * 64, 64)])
    @pl.when(s == 0)
    def _zero_counts():
        for b in range(5):
            pltpu.sync_copy(zcnt, cnt_sh.at[pl.ds(b * 1024, 1024)])

    plsc.subcore_barrier()

    pltpu.sync_copy(stage_sh, redv)
    fhi_g = jnp.int32(NBROWS)
    fge_g = jnp.int32(NBROWS)
    for t in range(NS):
        v = redv[t, pl.ds(0, 16)]
        fhi_g = jnp.minimum(fhi_g, v[0])
        fge_g = jnp.minimum(fge_g, v[1])
    p1 = fhi_g // 2
    p0 = (fge_g + 1) // 2
    corestart = jnp.where(c == 0, 0, p1)
    ltotal = jnp.where(c == 0, p0, NCHUNKS - p1)

    q = ltotal // NS
    r_ = ltotal - q * NS
    start = corestart + q * s + jnp.minimum(s, r_)
    cnt = q + jnp.where(s < r_, 1, 0)
    off = c * HALF

    lbase = jnp.minimum(start * BROWS, NBROWS - 160)
    pltpu.sync_copy(b_hbm.at[pl.ds(lbase, 160)], sbuf)

    def load(cid, buf, sx):
        cid = jnp.minimum(cid, NCHUNKS - 1)
        pltpu.async_copy(x_hbm.at[pl.ds(cid * CHUNK, CHUNK)], buf, sx)

    def wait(buf, sx):
        pltpu.make_async_copy(x_hbm.at[pl.ds(0, CHUNK)], buf, sx).wait()

    def scatter(cid, buf, rmp, so=None):
        dj = cid * BROWS - lbase
        for j in range(BROWS):
            for k in range(8):
                iv = sbuf[dj + j, pl.ds(k * 16, 16)] - off
                ok = (iv >= 0) & (iv < HALF)
                rmp[j, pl.ds(k * 16, 16)] = jnp.where(ok, iv, TRASH)
        descs = []
        for j in range(BROWS):
            pltpu.sync_copy(buf.at[pl.ds(j * 128, 128)],
                            acc_sh.at[rmp.at[j]], add=True)
            if so is None:
                pltpu.sync_copy(ones_v, cnt_sh.at[rmp.at[j]], add=True)
            else:
                descs.append(pltpu.async_copy(ones_v, cnt_sh.at[rmp.at[j]],
                                              so, add=True))
        return descs

    load(start, buf_a, sem_ax)
    load(start + 1, buf_b, sem_bx)

    def pair_body(i, _):
        ca = start + 2 * i
        wait(buf_a, sem_ax)
        da = scatter(ca, buf_a, rmp_a, sem_ao)
        load(ca + 2, buf_a, sem_ax)
        wait(buf_b, sem_bx)
        db = scatter(ca + 1, buf_b, rmp_b, sem_bo)
        load(ca + 3, buf_b, sem_bx)
        for d in da + db:
            d.wait()
        return 0
    lax.fori_loop(0, cnt // 2, pair_body, 0)
    wait(buf_a, sem_ax)
    wait(buf_b, sem_bx)

    @pl.when(cnt % 2 == 1)
    def _odd_tail():
        scatter(start + cnt - 1, buf_a, rmp_a)

    plsc.subcore_barrier()

    rbase = s * DIVQ

    def div_block(rb, nrows_blk):
        pltpu.sync_copy(acc_sh.at[pl.ds(rb, nrows_blk)],
                        buf_a.at[pl.ds(0, nrows_blk)])
        pltpu.sync_copy(cnt_sh.at[pl.ds(rb, nrows_blk)],
                        cntb.at[pl.ds(0, nrows_blk)])

        def row(rr, _):
            cv = cntb[pl.ds(rr, 16)]
            rec16 = jnp.ones((16,), jnp.float32) / jnp.maximum(cv, 1.0)
            rec = rec16[0]
            for j in range(8):
                buf_a[rr, pl.ds(j * 16, 16)] = buf_a[rr, pl.ds(j * 16, 16)] * rec
            return 0
        lax.fori_loop(0, nrows_blk, row, 0)
        pltpu.sync_copy(buf_a.at[pl.ds(0, nrows_blk)],
                        out_hbm.at[pl.ds(off + rb, nrows_blk)])

    for b in range(3):
        div_block(rbase + b * 104, 104)

    @pl.when(s == NS - 1)
    def _div_tail():
        div_block(rbase + 312, 8)


_sc_pool = pl.kernel(
    _sc_body,
    out_type=jax.ShapeDtypeStruct((S, D), jnp.float32),
    mesh=plsc.VectorSubcoreMesh(core_axis_name="c", subcore_axis_name="s"),
    scratch_types=[
        pltpu.VMEM((CHUNK, D), jnp.float32),
        pltpu.VMEM((CHUNK, D), jnp.float32),
        pltpu.VMEM((BROWS, 128), jnp.int32),
        pltpu.VMEM((BROWS, 128), jnp.int32),
        pltpu.VMEM((128,), jnp.float32),
        pltpu.VMEM((1024,), jnp.float32),
        pltpu.VMEM((160, 128), jnp.int32),
        pltpu.VMEM((128,), jnp.float32),
        pltpu.VMEM((NS, 16), jnp.int32),
        pltpu.VMEM((16,), jnp.int32),
        pltpu.VMEM_SHARED((ACC_R, D), jnp.float32),
        pltpu.VMEM_SHARED((ACC_R,), jnp.float32),
        pltpu.VMEM_SHARED((NS, 16), jnp.int32),
        pltpu.SemaphoreType.DMA,
        pltpu.SemaphoreType.DMA,
        pltpu.SemaphoreType.DMA,
        pltpu.SemaphoreType.DMA,
    ],
    compiler_params=pltpu.CompilerParams(use_tc_tiling_on_sc=False,
                                         needs_layout_passes=False),
)


@jax.jit
def kernel(x, batch):
    return _sc_pool(x, batch.reshape(NBROWS, 128))


# speedup vs baseline: 1.0295x; 1.0295x over previous
import jax
import jax.numpy as jnp
from jax import lax
from jax.experimental import pallas as pl
from jax.experimental.pallas import tpu as pltpu
from jax.experimental.pallas import tpu_sc as plsc

N = 320000
D = 128
S = 10000

NC = 2
NS = 16
HALF = S // NC
CHUNK = 256
BROWS = CHUNK // 128
NCHUNKS = N // CHUNK
NBROWS = N // 128
ACC_R = 5120
TRASH = HALF
SCANQ = NBROWS // NS
SCANR = NBROWS - SCANQ * NS
DIVQ = 312


def _sc_body(x_hbm, b_hbm, out_hbm,
             buf_a, buf_b, idx_a, idx_b, rmp_a, rmp_b, ones_v, zcnt,
             sbuf, cntb, redv, pubv,
             acc_sh, cnt_sh, stage_sh, sem_ax, sem_ai, sem_bx, sem_bi,
             sem_ao, sem_bo):
    c = lax.axis_index("c")
    s = lax.axis_index("s")
    z16 = jnp.zeros((16,), jnp.float32)
    lane = lax.iota(jnp.int32, 16)

    rowbase = s * SCANQ + jnp.minimum(s, SCANR)
    scan_dma = pltpu.async_copy(b_hbm.at[pl.ds(rowbase, SCANQ)],
                                sbuf.at[pl.ds(0, SCANQ)], sem_ax)

    for j in range(8):
        ones_v[pl.ds(j * 16, 16)] = jnp.ones((16,), jnp.float32)

    def zrow(i, _):
        for j in range(8):
            buf_a[i, pl.ds(j * 16, 16)] = z16
        return 0
    lax.fori_loop(0, 64, zrow, 0)

    def zc(i, _):
        zcnt[pl.ds(i * 16, 16)] = z16
        return 0
    lax.fori_loop(0, 64, zc, 0)

    scan_dma.wait()

    @pl.when(s < SCANR)
    def _extra_row():
        pltpu.sync_copy(b_hbm.at[pl.ds(rowbase + SCANQ, 1)],
                        sbuf.at[pl.ds(SCANQ, 1)])

    def scan_row(r, carry):
        fhi, fge = carry
        g = rowbase + r
        vhi = sbuf[r, pl.ds(112, 16)]
        vlo = sbuf[r, pl.ds(0, 16)]
        has_hi = jnp.any(vhi >= HALF)
        all_ge = jnp.all(vlo >= HALF)
        fhi = jnp.where(has_hi & (g < fhi), g, fhi)
        fge = jnp.where(all_ge & (g < fge), g, fge)
        return fhi, fge

    nrows = SCANQ + jnp.where(s < SCANR, 1, 0)
    fhi, fge = lax.fori_loop(0, nrows, scan_row,
                             (jnp.int32(NBROWS), jnp.int32(NBROWS)))
    pubv[...] = jnp.where(lane == 0, fhi, jnp.where(lane == 1, fge, NBROWS))
    pltpu.sync_copy(pubv, stage_sh.at[s])

    for b in range(5):
        pltpu.sync_copy(buf_a.at[pl.ds(0, 64)],
                        acc_sh.at[pl.ds(s * 320 + b * 64, 64)])
    @pl.when(s == 0)
    def _zero_counts():
        for b in range(5):
            pltpu.sync_copy(zcnt, cnt_sh.at[pl.ds(b * 1024, 1024)])

    plsc.subcore_barrier()

    pltpu.sync_copy(stage_sh, redv)
    fhi_g = jnp.int32(NBROWS)
    fge_g = jnp.int32(NBROWS)
    for t in range(NS):
        v = redv[t, pl.ds(0, 16)]
        fhi_g = jnp.minimum(fhi_g, v[0])
        fge_g = jnp.minimum(fge_g, v[1])
    p1 = fhi_g // 2
    p0 = (fge_g + 1) // 2
    corestart = jnp.where(c == 0, 0, p1)
    ltotal = jnp.where(c == 0, p0, NCHUNKS - p1)

    q = ltotal // NS
    r_ = ltotal - q * NS
    start = corestart + q * s + jnp.minimum(s, r_)
    cnt = q + jnp.where(s < r_, 1, 0)
    off = c * HALF

    def load(cid, buf, idx, sx, si):
        cid = jnp.minimum(cid, NCHUNKS - 1)
        pltpu.async_copy(x_hbm.at[pl.ds(cid * CHUNK, CHUNK)], buf, sx)
        pltpu.async_copy(b_hbm.at[pl.ds(cid * BROWS, BROWS)], idx, si)

    def wait(buf, idx, sx, si):
        pltpu.make_async_copy(x_hbm.at[pl.ds(0, CHUNK)], buf, sx).wait()
        pltpu.make_async_copy(b_hbm.at[pl.ds(0, BROWS)], idx, si).wait()

    def scatter(buf, idx, rmp, so=None):
        for j in range(BROWS):
            for k in range(8):
                iv = idx[j, pl.ds(k * 16, 16)] - off
                ok = (iv >= 0) & (iv < HALF)
                rmp[j, pl.ds(k * 16, 16)] = jnp.where(ok, iv, TRASH)
        descs = []
        for j in range(BROWS):
            pltpu.sync_copy(buf.at[pl.ds(j * 128, 128)],
                            acc_sh.at[rmp.at[j]], add=True)
            if so is None:
                pltpu.sync_copy(ones_v, cnt_sh.at[rmp.at[j]], add=True)
            else:
                descs.append(pltpu.async_copy(ones_v, cnt_sh.at[rmp.at[j]],
                                              so, add=True))
        return descs

    load(start, buf_a, idx_a, sem_ax, sem_ai)
    load(start + 1, buf_b, idx_b, sem_bx, sem_bi)

    def pair_body(i, _):
        ca = start + 2 * i
        wait(buf_a, idx_a, sem_ax, sem_ai)
        da = scatter(buf_a, idx_a, rmp_a, sem_ao)
        load(ca + 2, buf_a, idx_a, sem_ax, sem_ai)
        wait(buf_b, idx_b, sem_bx, sem_bi)
        db = scatter(buf_b, idx_b, rmp_b, sem_bo)
        load(ca + 3, buf_b, idx_b, sem_bx, sem_bi)
        for d in da + db:
            d.wait()
        return 0
    lax.fori_loop(0, cnt // 2, pair_body, 0)
    wait(buf_a, idx_a, sem_ax, sem_ai)
    wait(buf_b, idx_b, sem_bx, sem_bi)

    @pl.when(cnt % 2 == 1)
    def _odd_tail():
        scatter(buf_a, idx_a, rmp_a)

    plsc.subcore_barrier()

    rbase = s * DIVQ

    def div_block(rb, nunits):
        nrows_blk = nunits * 8
        pltpu.sync_copy(acc_sh.at[pl.ds(rb, nrows_blk)],
                        buf_a.at[pl.ds(0, nrows_blk)])
        pltpu.sync_copy(cnt_sh.at[pl.ds(rb, nrows_blk)],
                        cntb.at[pl.ds(0, nrows_blk)])

        def unit(u, _):
            cv = cntb[pl.ds(u * 8, 16)]
            rec16 = jnp.ones((16,), jnp.float32) / jnp.maximum(cv, 1.0)
            for k in range(8):
                rec = rec16[k]
                rr = u * 8 + k
                for j in range(8):
                    buf_a[rr, pl.ds(j * 16, 16)] = (
                        buf_a[rr, pl.ds(j * 16, 16)] * rec)
            return 0
        lax.fori_loop(0, nunits, unit, 0)
        pltpu.sync_copy(buf_a.at[pl.ds(0, nrows_blk)],
                        out_hbm.at[pl.ds(off + rb, nrows_blk)])

    for b in range(3):
        div_block(rbase + b * 104, 13)

    @pl.when(s == NS - 1)
    def _div_tail():
        div_block(rbase + 312, 1)


_sc_pool = pl.kernel(
    _sc_body,
    out_type=jax.ShapeDtypeStruct((S, D), jnp.float32),
    mesh=plsc.VectorSubcoreMesh(core_axis_name="c", subcore_axis_name="s"),
    scratch_types=[
        pltpu.VMEM((CHUNK, D), jnp.float32),
        pltpu.VMEM((CHUNK, D), jnp.float32),
        pltpu.VMEM((BROWS, 128), jnp.int32),
        pltpu.VMEM((BROWS, 128), jnp.int32),
        pltpu.VMEM((BROWS, 128), jnp.int32),
        pltpu.VMEM((BROWS, 128), jnp.int32),
        pltpu.VMEM((128,), jnp.float32),
        pltpu.VMEM((1024,), jnp.float32),
        pltpu.VMEM((SCANQ + 1, 128), jnp.int32),
        pltpu.VMEM((128,), jnp.float32),
        pltpu.VMEM((NS, 16), jnp.int32),
        pltpu.VMEM((16,), jnp.int32),
        pltpu.VMEM_SHARED((ACC_R, D), jnp.float32),
        pltpu.VMEM_SHARED((ACC_R,), jnp.float32),
        pltpu.VMEM_SHARED((NS, 16), jnp.int32),
        pltpu.SemaphoreType.DMA,
        pltpu.SemaphoreType.DMA,
        pltpu.SemaphoreType.DMA,
        pltpu.SemaphoreType.DMA,
        pltpu.SemaphoreType.DMA,
        pltpu.SemaphoreType.DMA,
    ],
    compiler_params=pltpu.CompilerParams(use_tc_tiling_on_sc=False,
                                         needs_layout_passes=False),
)


@jax.jit
def kernel(x, batch):
    return _sc_pool(x, batch.reshape(NBROWS, 128))


# speedup vs baseline: 1.0351x; 1.0054x over previous
import jax
import jax.numpy as jnp
from jax import lax
from jax.experimental import pallas as pl
from jax.experimental.pallas import tpu as pltpu
from jax.experimental.pallas import tpu_sc as plsc

N = 320000
D = 128
S = 10000

NC = 2
NS = 16
HALF = S // NC
CHUNK = 256
BROWS = CHUNK // 128
NCHUNKS = N // CHUNK
NBROWS = N // 128
ACC_R = 5120
TRASH = HALF
SCANQ = NBROWS // NS
SCANR = NBROWS - SCANQ * NS
DIVQ = 312


def _sc_body(x_hbm, b_hbm, out_hbm,
             buf_a, buf_b, idx_a, idx_b, rmp_a, rmp_b, ones_v, zcnt,
             sbuf, cntb, redv, pubv,
             acc_sh, cnt_sh, stage_sh, sem_ax, sem_ai, sem_bx, sem_bi,
             sem_ao, sem_bo):
    c = lax.axis_index("c")
    s = lax.axis_index("s")
    z16 = jnp.zeros((16,), jnp.float32)
    lane = lax.iota(jnp.int32, 16)

    rowbase = s * SCANQ + jnp.minimum(s, SCANR)
    scan_dma = pltpu.async_copy(b_hbm.at[pl.ds(rowbase, SCANQ)],
                                sbuf.at[pl.ds(0, SCANQ)], sem_ax)

    for j in range(8):
        ones_v[pl.ds(j * 16, 16)] = jnp.ones((16,), jnp.float32)

    def zrow(i, _):
        for j in range(8):
            buf_a[i, pl.ds(j * 16, 16)] = z16
        return 0
    lax.fori_loop(0, 64, zrow, 0)

    def zc(i, _):
        zcnt[pl.ds(i * 16, 16)] = z16
        return 0
    lax.fori_loop(0, 64, zc, 0)

    scan_dma.wait()

    @pl.when(s < SCANR)
    def _extra_row():
        pltpu.sync_copy(b_hbm.at[pl.ds(rowbase + SCANQ, 1)],
                        sbuf.at[pl.ds(SCANQ, 1)])

    def scan_row(r, carry):
        fhi, fge = carry
        g = rowbase + r
        vhi = sbuf[r, pl.ds(112, 16)]
        vlo = sbuf[r, pl.ds(0, 16)]
        has_hi = jnp.any(vhi >= HALF)
        all_ge = jnp.all(vlo >= HALF)
        fhi = jnp.where(has_hi & (g < fhi), g, fhi)
        fge = jnp.where(all_ge & (g < fge), g, fge)
        return fhi, fge

    nrows = SCANQ + jnp.where(s < SCANR, 1, 0)
    fhi, fge = lax.fori_loop(0, nrows, scan_row,
                             (jnp.int32(NBROWS), jnp.int32(NBROWS)))
    pubv[...] = jnp.where(lane == 0, fhi, jnp.where(lane == 1, fge, NBROWS))
    pltpu.sync_copy(pubv, stage_sh.at[s])

    for b in range(5):
        pltpu.sync_copy(buf_a.at[pl.ds(0, 64)],
                        acc_sh.at[pl.ds(s * 320 + b * 64, 64)])
    @pl.when(s == 0)
    def _zero_counts():
        for b in range(5):
            pltpu.sync_copy(zcnt, cnt_sh.at[pl.ds(b * 1024, 1024)])

    plsc.subcore_barrier()

    pltpu.sync_copy(stage_sh, redv)
    fhi_g = jnp.int32(NBROWS)
    fge_g = jnp.int32(NBROWS)
    for t in range(NS):
        v = redv[t, pl.ds(0, 16)]
        fhi_g = jnp.minimum(fhi_g, v[0])
        fge_g = jnp.minimum(fge_g, v[1])
    p1 = fhi_g // 2
    p0 = (fge_g + 1) // 2
    corestart = jnp.where(c == 0, 0, p1)
    ltotal = jnp.where(c == 0, p0, NCHUNKS - p1)

    q = ltotal // NS
    r_ = ltotal - q * NS
    start = corestart + q * s + jnp.minimum(s, r_)
    cnt = q + jnp.where(s < r_, 1, 0)
    off = c * HALF

    def load(cid, buf, idx, sx, si):
        cid = jnp.minimum(cid, NCHUNKS - 1)
        pltpu.async_copy(x_hbm.at[pl.ds(cid * CHUNK, CHUNK)], buf, sx)
        pltpu.async_copy(b_hbm.at[pl.ds(cid * BROWS, BROWS)], idx, si)

    def wait(buf, idx, sx, si):
        pltpu.make_async_copy(x_hbm.at[pl.ds(0, CHUNK)], buf, sx).wait()
        pltpu.make_async_copy(b_hbm.at[pl.ds(0, BROWS)], idx, si).wait()

    def scatter(buf, idx, rmp, so=None):
        for j in range(BROWS):
            for k in range(8):
                iv = idx[j, pl.ds(k * 16, 16)] - off
                ok = (iv >= 0) & (iv < HALF)
                rmp[j, pl.ds(k * 16, 16)] = jnp.where(ok, iv, TRASH)
        descs = []
        for j in range(BROWS):
            pltpu.sync_copy(buf.at[pl.ds(j * 128, 128)],
                            acc_sh.at[rmp.at[j]], add=True)
            if so is None:
                pltpu.sync_copy(ones_v, cnt_sh.at[rmp.at[j]], add=True)
            else:
                descs.append(pltpu.async_copy(ones_v, cnt_sh.at[rmp.at[j]],
                                              so, add=True))
        return descs

    load(start, buf_a, idx_a, sem_ax, sem_ai)
    load(start + 1, buf_b, idx_b, sem_bx, sem_bi)

    def pair_body(i, _):
        ca = start + 2 * i
        wait(buf_a, idx_a, sem_ax, sem_ai)
        da = scatter(buf_a, idx_a, rmp_a, sem_ao)
        load(ca + 2, buf_a, idx_a, sem_ax, sem_ai)
        wait(buf_b, idx_b, sem_bx, sem_bi)
        db = scatter(buf_b, idx_b, rmp_b, sem_bo)
        load(ca + 3, buf_b, idx_b, sem_bx, sem_bi)
        for d in da + db:
            d.wait()
        return 0
    lax.fori_loop(0, cnt // 2, pair_body, 0)
    wait(buf_a, idx_a, sem_ax, sem_ai)
    wait(buf_b, idx_b, sem_bx, sem_bi)

    @pl.when(cnt % 2 == 1)
    def _odd_tail():
        scatter(buf_a, idx_a, rmp_a)

    plsc.subcore_barrier()

    rbase = s * DIVQ

    pltpu.sync_copy(cnt_sh.at[pl.ds(rbase, 320)], cntb.at[pl.ds(0, 320)])

    def div_compute(buf, cb, nunits):
        def unit(u, _):
            cv = cntb[pl.ds(cb + u * 8, 16)]
            rec16 = jnp.ones((16,), jnp.float32) / jnp.maximum(cv, 1.0)
            for k in range(8):
                rec = rec16[k]
                rr = u * 8 + k
                for j in range(8):
                    buf[rr, pl.ds(j * 16, 16)] = (
                        buf[rr, pl.ds(j * 16, 16)] * rec)
            return 0
        lax.fori_loop(0, nunits, unit, 0)

    def dload(rb, buf, sx):
        return pltpu.async_copy(acc_sh.at[pl.ds(rb, 104)],
                                buf.at[pl.ds(0, 104)], sx)

    d0 = dload(rbase, buf_a, sem_ax)
    d1 = dload(rbase + 104, buf_b, sem_bx)
    d0.wait()
    div_compute(buf_a, 0, 13)
    pltpu.sync_copy(buf_a.at[pl.ds(0, 104)], out_hbm.at[pl.ds(off + rbase, 104)])
    d2 = dload(rbase + 208, buf_a, sem_ax)
    d1.wait()
    div_compute(buf_b, 104, 13)
    pltpu.sync_copy(buf_b.at[pl.ds(0, 104)],
                    out_hbm.at[pl.ds(off + rbase + 104, 104)])
    d2.wait()
    div_compute(buf_a, 208, 13)
    pltpu.sync_copy(buf_a.at[pl.ds(0, 104)],
                    out_hbm.at[pl.ds(off + rbase + 208, 104)])

    @pl.when(s == NS - 1)
    def _div_tail():
        pltpu.sync_copy(acc_sh.at[pl.ds(rbase + 312, 8)],
                        buf_b.at[pl.ds(0, 8)])
        div_compute(buf_b, 312, 1)
        pltpu.sync_copy(buf_b.at[pl.ds(0, 8)],
                        out_hbm.at[pl.ds(off + rbase + 312, 8)])


_sc_pool = pl.kernel(
    _sc_body,
    out_type=jax.ShapeDtypeStruct((S, D), jnp.float32),
    mesh=plsc.VectorSubcoreMesh(core_axis_name="c", subcore_axis_name="s"),
    scratch_types=[
        pltpu.VMEM((CHUNK, D), jnp.float32),
        pltpu.VMEM((CHUNK, D), jnp.float32),
        pltpu.VMEM((BROWS, 128), jnp.int32),
        pltpu.VMEM((BROWS, 128), jnp.int32),
        pltpu.VMEM((BROWS, 128), jnp.int32),
        pltpu.VMEM((BROWS, 128), jnp.int32),
        pltpu.VMEM((128,), jnp.float32),
        pltpu.VMEM((1024,), jnp.float32),
        pltpu.VMEM((SCANQ + 1, 128), jnp.int32),
        pltpu.VMEM((336,), jnp.float32),
        pltpu.VMEM((NS, 16), jnp.int32),
        pltpu.VMEM((16,), jnp.int32),
        pltpu.VMEM_SHARED((ACC_R, D), jnp.float32),
        pltpu.VMEM_SHARED((ACC_R,), jnp.float32),
        pltpu.VMEM_SHARED((NS, 16), jnp.int32),
        pltpu.SemaphoreType.DMA,
        pltpu.SemaphoreType.DMA,
        pltpu.SemaphoreType.DMA,
        pltpu.SemaphoreType.DMA,
        pltpu.SemaphoreType.DMA,
        pltpu.SemaphoreType.DMA,
    ],
    compiler_params=pltpu.CompilerParams(use_tc_tiling_on_sc=False,
                                         needs_layout_passes=False),
)


@jax.jit
def kernel(x, batch):
    return _sc_pool(x, batch.reshape(NBROWS, 128))
